# baseline (speedup 1.0000x reference)
import functools

import jax
import jax.numpy as jnp
from jax import lax
from jax.experimental import pallas as pl
from jax.experimental.pallas import tpu as pltpu
from jax.experimental.pallas import tpu_sc as plsc

N = 10000
NPAD = 10240
E = 320000
EPAD = 327680
NSC = 2
NTILE = 16
K = 80
ROWS_PER_TILE = NPAD // NTILE


def _f32(*shape):
    return jax.ShapeDtypeStruct(shape, jnp.float32)


def _make_sc_agg(with_deg, edge_split):
    NB = 4
    n_workers = NSC * NTILE if edge_split else NTILE
    per_tile = EPAD // n_workers
    ch = per_tile // K
    half = ch // 2
    S = 32
    n_super = ch // S
    SK = S * K

    n_tab = 1 if edge_split else 2
    n_in = n_tab + 2 + (1 if with_deg else 0)
    n_out = 2 + (1 if with_deg else 0)

    out_type = [_f32(NPAD, 128), _f32(NPAD, 128)]
    if with_deg:
        out_type.append(_f32(2 * NPAD))

    scratch = [
        pltpu.VMEM_SHARED((NPAD, 128), jnp.float32),
        pltpu.VMEM((SK,), jnp.int32),
        pltpu.VMEM((SK,), jnp.int32),
    ]
    scratch += [pltpu.VMEM((K, 128), jnp.float32) for _ in range(NB)]
    scratch += [pltpu.VMEM((K,), jnp.int32) for _ in range(NB)]
    scratch += [pltpu.SemaphoreType.DMA for _ in range(2 * NB)]
    if with_deg:
        scratch.append(pltpu.VMEM_SHARED((NPAD,), jnp.float32))
        scratch.append(pltpu.VMEM((K,), jnp.float32))

    def body(*refs):
        ins = refs[:n_in]
        outs = refs[n_in:n_in + n_out]
        scr = refs[n_in + n_out:]
        tabs = ins[:n_tab]
        src1 = ins[n_tab]
        dst1 = ins[n_tab + 1]
        out_a, out_b = outs[0], outs[1]

        acc_sh, sidx, didx = scr[0], scr[1], scr[2]
        rows = scr[3:3 + NB]
        dbuf = scr[3 + NB:3 + 2 * NB]
        gsem = scr[3 + 2 * NB:3 + 3 * NB]
        ssem = scr[3 + 3 * NB:3 + 4 * NB]
        if with_deg:
            zeros1d = ins[n_tab + 2]
            out_deg = outs[2]
            deg_sh = scr[3 + 4 * NB]
            ones_v = scr[3 + 4 * NB + 1]

        cid = lax.axis_index("c")
        sid = lax.axis_index("s")
        row0 = sid * ROWS_PER_TILE
        base = (cid * NTILE + sid) * per_tile if edge_split else sid * per_tile

        zrows = ROWS_PER_TILE // K
        def zfill(r, c):
            rows[0][r, pl.ds(c * 16, 16)] = jnp.zeros((16,), jnp.float32)
            return c
        lax.fori_loop(0, K, lambda r, c: lax.fori_loop(0, 8, lambda c2, _: zfill(r, c2), 0), 0)
        for z in range(zrows):
            pltpu.sync_copy(rows[0], acc_sh.at[pl.ds(row0 + z * K, K)])
        if with_deg:
            pltpu.sync_copy(zeros1d.at[pl.ds(row0, ROWS_PER_TILE)],
                            deg_sh.at[pl.ds(row0, ROWS_PER_TILE)])
            for j in range(K // 16):
                ones_v[pl.ds(j * 16, 16)] = jnp.ones((16,), jnp.float32)
        plsc.subcore_barrier()

        def gather(c, b):
            idx = sidx.at[pl.ds(c * K, K)]
            if edge_split:
                pltpu.async_copy(tabs[0].at[idx], rows[b], gsem[b])
            else:
                @pl.when(cid == 0)
                def _():
                    pltpu.async_copy(tabs[0].at[idx], rows[b], gsem[b])

                @pl.when(cid == 1)
                def _():
                    pltpu.async_copy(tabs[1].at[idx], rows[b], gsem[b])

        def gwait(b):
            pltpu.make_async_copy(tabs[0].at[sidx.at[pl.ds(0, K)]],
                                  rows[b], gsem[b]).wait()

        def deg_cond(jg):
            return lax.select(cid == 0, jg < half, jg >= half)

        def scat(c, jg, b):
            off = c * K
            for i in range(K // 16):
                dbuf[b][pl.ds(i * 16, 16)] = didx[pl.ds(off + i * 16, 16)]
            pltpu.async_copy(rows[b], acc_sh.at[dbuf[b]], ssem[b], add=True)
            if with_deg:
                @pl.when(deg_cond(jg))
                def _():
                    pltpu.async_copy(ones_v, deg_sh.at[dbuf[b]], ssem[b],
                                     add=True)

        def sdrain(jg, b):
            pltpu.make_async_copy(rows[b], acc_sh.at[dbuf[b]], ssem[b]).wait()
            if with_deg:
                @pl.when(deg_cond(jg))
                def _():
                    pltpu.make_async_copy(ones_v, deg_sh.at[dbuf[b]],
                                          ssem[b]).wait()

        def super_chunk(s, carry):
            sbase = base + s * SK
            pltpu.sync_copy(src1.at[pl.ds(sbase, SK)], sidx)
            pltpu.sync_copy(dst1.at[pl.ds(sbase, SK)], didx)
            jg0 = s * S

            gather(0, 0)
            gather(1, 1)

            def quad(t4, c4):
                for u in range(NB):
                    c = NB * t4 + u
                    jg = jg0 + c
                    gwait(u)
                    scat(c, jg, u)
                    b2 = (u + 2) % NB

                    @pl.when(c + 2 < S)
                    def _():
                        @pl.when(c >= 2)
                        def _():
                            sdrain(jg - 2, b2)
                        gather(c + 2, b2)

                return c4

            lax.fori_loop(0, S // NB, quad, 0)
            for d in range(NB):
                m = S - NB + d
                sdrain(jg0 + m, m % NB)
            return carry

        lax.fori_loop(0, n_super, super_chunk, 0)

        plsc.subcore_barrier()

        @pl.when(cid == 0)
        def _():
            pltpu.sync_copy(acc_sh.at[pl.ds(row0, ROWS_PER_TILE)],
                            out_a.at[pl.ds(row0, ROWS_PER_TILE)])

        @pl.when(cid == 1)
        def _():
            pltpu.sync_copy(acc_sh.at[pl.ds(row0, ROWS_PER_TILE)],
                            out_b.at[pl.ds(row0, ROWS_PER_TILE)])

        if with_deg:
            pltpu.sync_copy(deg_sh.at[pl.ds(row0, ROWS_PER_TILE)],
                            out_deg.at[pl.ds(cid * NPAD + row0, ROWS_PER_TILE)])

    mesh = plsc.VectorSubcoreMesh(core_axis_name="c", subcore_axis_name="s")
    name = "sc_agg_l2" if edge_split else "sc_agg_l1"
    return pl.kernel(body, out_type=tuple(out_type), mesh=mesh,
                     scratch_types=scratch, name=name)


_BLK = 2000
_GRID = N // _BLK


def _row_spec(r, c):
    return pl.BlockSpec((r, c), lambda i: (i, 0))


def _full_spec(r, c):
    return pl.BlockSpec((r, c), lambda i: (0, 0))


def _stage_a(ori, struc, Wn1o, Ws1o, b1o, Wn1s, Ws1s, b1s):
    def body(x_o, x_s, wno, wso, bo, wns, wss, bs, t_o, t_s, s_o, s_s):
        xo = x_o[...]
        xs = x_s[...]
        t_o[...] = jnp.dot(xo, wno[...], preferred_element_type=jnp.float32)
        t_s[...] = jnp.dot(xs, wns[...], preferred_element_type=jnp.float32)
        s_o[...] = jnp.dot(xo, wso[...], preferred_element_type=jnp.float32) + bo[...]
        s_s[...] = jnp.dot(xs, wss[...], preferred_element_type=jnp.float32) + bs[...]

    return pl.pallas_call(
        body,
        grid=(_GRID,),
        in_specs=[_row_spec(_BLK, 128), _row_spec(_BLK, 128),
                  _full_spec(128, 128), _full_spec(128, 128), _full_spec(1, 128),
                  _full_spec(128, 128), _full_spec(128, 128), _full_spec(1, 128)],
        out_specs=[_row_spec(_BLK, 128), _row_spec(_BLK, 128),
                   _row_spec(_BLK, 128), _row_spec(_BLK, 128)],
        out_shape=[_f32(NPAD, 128), _f32(NPAD, 128), _f32(N, 128), _f32(N, 128)],
    )(ori, struc, Wn1o, Ws1o, b1o.reshape(1, 128), Wn1s, Ws1s, b1s.reshape(1, 128))


def _stage_c(Ao, As, dega, degb, S1o, S1s, Wn2o, Ws2o, b2o, Wn2s, Ws2s, b2s):
    def body(a_o, a_s, d_a, d_b, s1o, s1s, wno, wso, bo, wns, wss, bs,
             tab2, s2):
        r = 1.0 / jnp.maximum(d_a[...] + d_b[...], 1.0)
        h_o = jax.nn.relu(s1o[...] + a_o[...] * r)
        h_s = jax.nn.relu(s1s[...] + a_s[...] * r)
        t_o = jnp.dot(h_o, wno[...], preferred_element_type=jnp.float32)
        t_s = jnp.dot(h_s, wns[...], preferred_element_type=jnp.float32)
        tab2[...] = jnp.concatenate([t_o, t_s], axis=1)
        so = jnp.dot(h_o, wso[...], preferred_element_type=jnp.float32) + bo[...]
        ss = jnp.dot(h_s, wss[...], preferred_element_type=jnp.float32) + bs[...]
        s2[...] = jnp.concatenate([so, ss], axis=1)

    return pl.pallas_call(
        body,
        grid=(_GRID,),
        in_specs=[_row_spec(_BLK, 128), _row_spec(_BLK, 128),
                  _row_spec(_BLK, 1), _row_spec(_BLK, 1),
                  _row_spec(_BLK, 128), _row_spec(_BLK, 128),
                  _full_spec(128, 64), _full_spec(128, 64), _full_spec(1, 64),
                  _full_spec(128, 64), _full_spec(128, 64), _full_spec(1, 64)],
        out_specs=[_row_spec(_BLK, 128), _row_spec(_BLK, 128)],
        out_shape=[_f32(NPAD, 128), _f32(N, 128)],
    )(Ao, As, dega, degb, S1o, S1s,
      Wn2o, Ws2o, b2o.reshape(1, 64), Wn2s, Ws2s, b2s.reshape(1, 64))


def _stage_e(A2p0, A2p1, dega, degb, S2, W1, b1, W2, b2):
    def body(a0, a1, d_a, d_b, s2, w1, bb1, w2, bb2, out):
        r = 1.0 / jnp.maximum(d_a[...] + d_b[...], 1.0)
        h2 = s2[...] + (a0[...] + a1[...]) * r
        z = jax.nn.relu(jnp.dot(h2, w1[...], preferred_element_type=jnp.float32)
                        + bb1[...])
        out[...] = jnp.dot(z, w2[...], preferred_element_type=jnp.float32) + bb2[...]

    return pl.pallas_call(
        body,
        grid=(_GRID,),
        in_specs=[_row_spec(_BLK, 128), _row_spec(_BLK, 128),
                  _row_spec(_BLK, 1), _row_spec(_BLK, 1),
                  _row_spec(_BLK, 128),
                  _full_spec(128, 128), _full_spec(1, 128),
                  _full_spec(128, 64), _full_spec(1, 64)],
        out_specs=[_row_spec(_BLK, 64)],
        out_shape=[_f32(N, 64)],
    )(A2p0, A2p1, dega, degb, S2, W1, b1.reshape(1, 128), W2, b2.reshape(1, 64))[0]


def kernel(ori_feat, struc_feat, edge_index, Ws1o, Wn1o, b1o, Ws2o, Wn2o, b2o,
           Ws1s, Wn1s, b1s, Ws2s, Wn2s, b2s, mlp_W1, mlp_b1, mlp_W2, mlp_b2):
    pad = EPAD - E
    pad_i = jnp.arange(pad, dtype=jnp.int32)
    src2 = jnp.concatenate([edge_index[0], pad_i % N])
    dst2 = jnp.concatenate([edge_index[1], N + pad_i % (NPAD - N)])
    zeros1d = jnp.zeros((NPAD,), jnp.float32)

    tab1o, tab1s, S1o, S1s = _stage_a(ori_feat, struc_feat,
                                      Wn1o, Ws1o, b1o, Wn1s, Ws1s, b1s)

    Ao, As, deg2 = _make_sc_agg(True, False)(
        tab1o, tab1s, src2, dst2, zeros1d)

    dega = deg2[:N].reshape(N, 1)
    degb = deg2[NPAD:NPAD + N].reshape(N, 1)

    tab2, S2 = _stage_c(Ao[:N], As[:N], dega, degb, S1o, S1s,
                        Wn2o, Ws2o, b2o, Wn2s, Ws2s, b2s)

    A2p0, A2p1 = _make_sc_agg(False, True)(tab2, src2, dst2)

    return _stage_e(A2p0[:N], A2p1[:N], dega, degb, S2,
                    mlp_W1, mlp_b1, mlp_W2, mlp_b2)


# speedup vs baseline: 1.0643x; 1.0643x over previous
import functools

import jax
import jax.numpy as jnp
from jax import lax
from jax.experimental import pallas as pl
from jax.experimental.pallas import tpu as pltpu
from jax.experimental.pallas import tpu_sc as plsc

N = 10000
NPAD = 10240
E = 320000
EPAD = 327680
NSC = 2
NTILE = 16
K = 128
ROWS_PER_TILE = NPAD // NTILE


DT = jnp.float32


def _f32(*shape):
    return jax.ShapeDtypeStruct(shape, jnp.float32)


def _make_sc_agg(with_deg, edge_split):
    n_workers = NSC * NTILE if edge_split else NTILE
    per_tile = EPAD // n_workers
    ch = per_tile // K
    half = ch // 2
    S = 32 if not edge_split else 40
    n_super = ch // S
    SK = S * K

    n_tab = 1 if edge_split else 2
    n_in = n_tab + 3 + (1 if with_deg else 0)
    n_out = 2 + (1 if with_deg else 0)

    out_type = [jax.ShapeDtypeStruct((NPAD, 128), DT), jax.ShapeDtypeStruct((NPAD, 128), DT)]
    if with_deg:
        out_type.append(_f32(2 * NPAD))

    scratch = [
        pltpu.VMEM_SHARED((NPAD, 128), DT),
        pltpu.VMEM((SK,), jnp.int32),
        pltpu.VMEM((SK,), jnp.int32),
        pltpu.VMEM((K,), jnp.int32),
        pltpu.VMEM((K, 128), DT),
        pltpu.VMEM((K, 128), DT),
        pltpu.SemaphoreType.DMA,
        pltpu.SemaphoreType.DMA,
    ]
    if with_deg:
        scratch.append(pltpu.VMEM_SHARED((NPAD,), jnp.float32))
        scratch.append(pltpu.VMEM((K,), jnp.float32))
        scratch.append(pltpu.SemaphoreType.DMA)

    def body(*refs):
        ins = refs[:n_in]
        outs = refs[n_in:n_in + n_out]
        scr = refs[n_in + n_out:]
        tabs = ins[:n_tab]
        src1, dst1, zeros2d = ins[n_tab:n_tab + 3]
        if with_deg:
            zeros1d = ins[n_tab + 3]
            out_deg = outs[2]
            (acc_sh, sidx, didx, dbuf, rows0, rows1, sem0, sem1,
             deg_sh, ones_v, dsem) = scr
        else:
            acc_sh, sidx, didx, dbuf, rows0, rows1, sem0, sem1 = scr
        out_a, out_b = outs[0], outs[1]

        cid = lax.axis_index("c")
        sid = lax.axis_index("s")
        row0 = sid * ROWS_PER_TILE
        base = (cid * NTILE + sid) * per_tile if edge_split else sid * per_tile

        pltpu.sync_copy(zeros2d.at[pl.ds(row0, ROWS_PER_TILE)],
                        acc_sh.at[pl.ds(row0, ROWS_PER_TILE)])
        if with_deg:
            pltpu.sync_copy(zeros1d.at[pl.ds(row0, ROWS_PER_TILE)],
                            deg_sh.at[pl.ds(row0, ROWS_PER_TILE)])
            for j in range(K // 16):
                ones_v[pl.ds(j * 16, 16)] = jnp.ones((16,), jnp.float32)
        plsc.subcore_barrier()

        def gather(q, buf, sem):
            idx = sidx.at[pl.ds(q * K, K)]
            if edge_split:
                pltpu.async_copy(tabs[0].at[idx], buf, sem)
            else:
                @pl.when(cid == 0)
                def _():
                    pltpu.async_copy(tabs[0].at[idx], buf, sem)

                @pl.when(cid == 1)
                def _():
                    pltpu.async_copy(tabs[1].at[idx], buf, sem)

        def gwait(buf, sem):
            pltpu.make_async_copy(tabs[0].at[sidx.at[pl.ds(0, K)]],
                                  buf, sem).wait()

        def deg_cond(jglob):
            return lax.select(cid == 0, jglob < half, jglob >= half)

        def scatter(q, jglob, buf):
            if with_deg:
                @pl.when(jnp.logical_and(jglob >= 1, deg_cond(jglob - 1)))
                def _():
                    pltpu.make_async_copy(ones_v, deg_sh.at[dbuf], dsem).wait()
            off = q * K
            for i in range(K // 16):
                dbuf[pl.ds(i * 16, 16)] = didx[pl.ds(off + i * 16, 16)]
            pltpu.sync_copy(buf, acc_sh.at[dbuf], add=True)
            if with_deg:
                @pl.when(deg_cond(jglob))
                def _():
                    pltpu.async_copy(ones_v, deg_sh.at[dbuf], dsem, add=True)

        def super_chunk(s, carry):
            sbase = base + s * SK
            pltpu.sync_copy(src1.at[pl.ds(sbase, SK)], sidx)
            pltpu.sync_copy(dst1.at[pl.ds(sbase, SK)], didx)

            gather(0, rows0, sem0)
            gather(1, rows1, sem1)

            def pair(q2, c2):
                a = 2 * q2
                gwait(rows0, sem0)
                scatter(a, s * S + a, rows0)

                @pl.when(a + 2 < S)
                def _():
                    gather(a + 2, rows0, sem0)

                gwait(rows1, sem1)
                scatter(a + 1, s * S + a + 1, rows1)

                @pl.when(a + 3 < S)
                def _():
                    gather(a + 3, rows1, sem1)

                return c2

            lax.fori_loop(0, S // 2, pair, 0)
            return carry

        lax.fori_loop(0, n_super, super_chunk, 0)
        if with_deg:
            @pl.when(deg_cond(ch - 1))
            def _():
                pltpu.make_async_copy(ones_v, deg_sh.at[dbuf], dsem).wait()

        plsc.subcore_barrier()

        @pl.when(cid == 0)
        def _():
            pltpu.sync_copy(acc_sh.at[pl.ds(row0, ROWS_PER_TILE)],
                            out_a.at[pl.ds(row0, ROWS_PER_TILE)])

        @pl.when(cid == 1)
        def _():
            pltpu.sync_copy(acc_sh.at[pl.ds(row0, ROWS_PER_TILE)],
                            out_b.at[pl.ds(row0, ROWS_PER_TILE)])

        if with_deg:
            pltpu.sync_copy(deg_sh.at[pl.ds(row0, ROWS_PER_TILE)],
                            out_deg.at[pl.ds(cid * NPAD + row0, ROWS_PER_TILE)])

    mesh = plsc.VectorSubcoreMesh(core_axis_name="c", subcore_axis_name="s")
    name = "sc_agg_l2" if edge_split else "sc_agg_l1"
    return pl.kernel(body, out_type=tuple(out_type), mesh=mesh,
                     scratch_types=scratch, name=name)


_BLK = 2000
_GRID = N // _BLK


def _row_spec(r, c):
    return pl.BlockSpec((r, c), lambda i: (i, 0))


def _full_spec(r, c):
    return pl.BlockSpec((r, c), lambda i: (0, 0))


def _stage_a(ori, struc, Wn1o, Ws1o, b1o, Wn1s, Ws1s, b1s):
    def body(x_o, x_s, wno, wso, bo, wns, wss, bs, t_o, t_s, s_o, s_s):
        xo = x_o[...]
        xs = x_s[...]
        t_o[...] = jnp.dot(xo, wno[...],
                           preferred_element_type=jnp.float32).astype(DT)
        t_s[...] = jnp.dot(xs, wns[...],
                           preferred_element_type=jnp.float32).astype(DT)
        s_o[...] = jnp.dot(xo, wso[...], preferred_element_type=jnp.float32) + bo[...]
        s_s[...] = jnp.dot(xs, wss[...], preferred_element_type=jnp.float32) + bs[...]

    return pl.pallas_call(
        body,
        grid=(_GRID,),
        in_specs=[_row_spec(_BLK, 128), _row_spec(_BLK, 128),
                  _full_spec(128, 128), _full_spec(128, 128), _full_spec(1, 128),
                  _full_spec(128, 128), _full_spec(128, 128), _full_spec(1, 128)],
        out_specs=[_row_spec(_BLK, 128), _row_spec(_BLK, 128),
                   _row_spec(_BLK, 128), _row_spec(_BLK, 128)],
        out_shape=[jax.ShapeDtypeStruct((NPAD, 128), DT),
                   jax.ShapeDtypeStruct((NPAD, 128), DT),
                   _f32(N, 128), _f32(N, 128)],
    )(ori, struc, Wn1o, Ws1o, b1o.reshape(1, 128), Wn1s, Ws1s, b1s.reshape(1, 128))


def _stage_c(Ao, As, dega, degb, S1o, S1s, Wn2o, Ws2o, b2o, Wn2s, Ws2s, b2s):
    def body(a_o, a_s, d_a, d_b, s1o, s1s, wno, wso, bo, wns, wss, bs,
             tab2, s2):
        r = 1.0 / jnp.maximum(d_a[...] + d_b[...], 1.0)
        h_o = jax.nn.relu(s1o[...] + a_o[...].astype(jnp.float32) * r)
        h_s = jax.nn.relu(s1s[...] + a_s[...].astype(jnp.float32) * r)
        t_o = jnp.dot(h_o, wno[...], preferred_element_type=jnp.float32)
        t_s = jnp.dot(h_s, wns[...], preferred_element_type=jnp.float32)
        tab2[...] = jnp.concatenate([t_o, t_s], axis=1).astype(DT)
        so = jnp.dot(h_o, wso[...], preferred_element_type=jnp.float32) + bo[...]
        ss = jnp.dot(h_s, wss[...], preferred_element_type=jnp.float32) + bs[...]
        s2[...] = jnp.concatenate([so, ss], axis=1)

    return pl.pallas_call(
        body,
        grid=(_GRID,),
        in_specs=[_row_spec(_BLK, 128), _row_spec(_BLK, 128),
                  _row_spec(_BLK, 1), _row_spec(_BLK, 1),
                  _row_spec(_BLK, 128), _row_spec(_BLK, 128),
                  _full_spec(128, 64), _full_spec(128, 64), _full_spec(1, 64),
                  _full_spec(128, 64), _full_spec(128, 64), _full_spec(1, 64)],
        out_specs=[_row_spec(_BLK, 128), _row_spec(_BLK, 128)],
        out_shape=[jax.ShapeDtypeStruct((NPAD, 128), DT), _f32(N, 128)],
    )(Ao, As, dega, degb, S1o, S1s,
      Wn2o, Ws2o, b2o.reshape(1, 64), Wn2s, Ws2s, b2s.reshape(1, 64))


def _stage_e(A2p0, A2p1, dega, degb, S2, W1, b1, W2, b2):
    def body(a0, a1, d_a, d_b, s2, w1, bb1, w2, bb2, out):
        r = 1.0 / jnp.maximum(d_a[...] + d_b[...], 1.0)
        h2 = s2[...] + (a0[...].astype(jnp.float32)
                        + a1[...].astype(jnp.float32)) * r
        z = jax.nn.relu(jnp.dot(h2, w1[...], preferred_element_type=jnp.float32)
                        + bb1[...])
        out[...] = jnp.dot(z, w2[...], preferred_element_type=jnp.float32) + bb2[...]

    return pl.pallas_call(
        body,
        grid=(_GRID,),
        in_specs=[_row_spec(_BLK, 128), _row_spec(_BLK, 128),
                  _row_spec(_BLK, 1), _row_spec(_BLK, 1),
                  _row_spec(_BLK, 128),
                  _full_spec(128, 128), _full_spec(1, 128),
                  _full_spec(128, 64), _full_spec(1, 64)],
        out_specs=[_row_spec(_BLK, 64)],
        out_shape=[_f32(N, 64)],
    )(A2p0, A2p1, dega, degb, S2, W1, b1.reshape(1, 128), W2, b2.reshape(1, 64))[0]


def kernel(ori_feat, struc_feat, edge_index, Ws1o, Wn1o, b1o, Ws2o, Wn2o, b2o,
           Ws1s, Wn1s, b1s, Ws2s, Wn2s, b2s, mlp_W1, mlp_b1, mlp_W2, mlp_b2):
    pad = EPAD - E
    pad_i = jnp.arange(pad, dtype=jnp.int32)
    src2 = jnp.concatenate([edge_index[0], pad_i % N])
    dst2 = jnp.concatenate([edge_index[1], N + pad_i % (NPAD - N)])
    zeros2d = jnp.zeros((NPAD, 128), DT)
    zeros1d = jnp.zeros((NPAD,), jnp.float32)

    tab1o, tab1s, S1o, S1s = _stage_a(ori_feat, struc_feat,
                                      Wn1o, Ws1o, b1o, Wn1s, Ws1s, b1s)

    Ao, As, deg2 = _make_sc_agg(True, False)(
        tab1o, tab1s, src2, dst2, zeros2d, zeros1d)

    dega = deg2[:N].reshape(N, 1)
    degb = deg2[NPAD:NPAD + N].reshape(N, 1)

    tab2, S2 = _stage_c(Ao[:N], As[:N], dega, degb, S1o, S1s,
                        Wn2o, Ws2o, b2o, Wn2s, Ws2s, b2s)

    A2p0, A2p1 = _make_sc_agg(False, True)(tab2, src2, dst2, zeros2d)

    return _stage_e(A2p0[:N], A2p1[:N], dega, degb, S2,
                    mlp_W1, mlp_b1, mlp_W2, mlp_b2)
